# baseline (speedup 1.0000x reference)
import jax
import jax.numpy as jnp
from jax import lax
from jax.experimental import pallas as pl
from jax.experimental.pallas import tpu as pltpu

B = 64
C = 2048
V = 256
NSLOT = 30
SPAD = 32
T = 256
NB = C // T

_U32 = jnp.uint32


def _tf(ka, kb, x0c, x1c):
    ks2 = ka ^ kb ^ _U32(0x1BD11BDA)
    x0 = ka + _U32(x0c)
    x1 = kb + _U32(x1c)
    rot = (13, 15, 26, 6, 17, 29, 16, 24)
    ks = (kb, ks2, ka)
    for i in range(5):
        for j in range(4):
            r = rot[(i % 2) * 4 + j]
            x0 = x0 + x1
            x1 = ((x1 << _U32(r)) | (x1 >> _U32(32 - r))) ^ x0
        x0 = x0 + ks[i % 3]
        x1 = x1 + ks[(i + 1) % 3] + _U32(i + 1)
    return x0, x1


def _mod10(x_u32):
    y = (x_u32 & _U32(0xFFFF)) + _U32(6) * (x_u32 >> _U32(16))
    yf = y.astype(jnp.float32)
    q = ((yf + 0.5) * 0.1).astype(jnp.int32)
    return y.astype(jnp.int32) - 10 * q


def _kernel(kd_ref,
            tokens_ref, probs_ref, imask_ref, tmask_ref,
            key_sc,
            keys_a, keys_b,
            tokr_sc, src_sc, off_sc,
            tokb_sc, tmb_sc,
            dists_sc, comb_sc,
            negb_sc):
    i = pl.program_id(0)

    rows = lax.broadcasted_iota(jnp.int32, (SPAD, B), 0)
    rows_real = rows < NSLOT

    @pl.when(i == 0)
    def _init():
        dists_sc[...] = jnp.where(rows_real, 0, -100000)
        comb_sc[...] = (1 << 10) | (rows << 11)
        negb_sc[...] = jnp.full((1, B), (1 << NSLOT) - 1, jnp.int32)

        def chain_body(t, carry):
            ka, kb = carry
            keys_a[t, :] = ka
            keys_b[t, :] = kb
            return _tf(ka, kb, 0, 3)

        kaf, kbf = lax.fori_loop(0, T, chain_body,
                                 (kd_ref[0, :], kd_ref[1, :]), unroll=4)
        key_sc[0, :] = kaf
        key_sc[1, :] = kbf

    ka = keys_a[...]
    kb = keys_b[...]
    k1a, k1b = _tf(ka, kb, 0, 0)
    k2a, k2b = _tf(ka, kb, 0, 1)
    k3a, k3b = _tf(ka, kb, 0, 2)
    s1a, s1b = _tf(k1a, k1b, 0, 1)
    la, lb = _tf(s1a, s1b, 0, 0)
    tokr_sc[...] = ((la ^ lb) & _U32(255)).astype(jnp.int32)
    s0a, s0b = _tf(k2a, k2b, 0, 0)
    s1a, s1b = _tf(k2a, k2b, 0, 1)
    ha, hb = _tf(s0a, s0b, 0, 0)
    la, lb = _tf(s1a, s1b, 0, 0)
    hi10 = _mod10(ha ^ hb)
    lo10 = _mod10(la ^ lb)
    v = hi10 * 6 + lo10
    vq = ((v.astype(jnp.float32) + 0.5) * 0.1).astype(jnp.int32)
    src_sc[...] = (v - 10 * vq == 0).astype(jnp.int32)
    s1a, s1b = _tf(k3a, k3b, 0, 1)
    la, lb = _tf(s1a, s1b, 0, 0)
    off_sc[...] = ((la ^ lb) & _U32(255)).astype(jnp.int32)
    bitw = jnp.where(rows_real, 1 << jnp.minimum(rows, 30), 0)
    REAL = (1 << NSLOT) - 1

    def step_body(t, carry):
        cka, ckb, negbits = carry
        keys_a[t, :] = cka
        keys_b[t, :] = ckb
        nka, nkb = _tf(cka, ckb, 0, 3)

        nb0 = negbits
        ub0 = (~nb0) & REAL
        s0 = nb0 != 0
        p0 = jnp.where(s0, nb0 & -nb0, ub0 & -ub0)
        nb1 = nb0 & ~p0
        ub1 = ub0 & ~p0
        s1 = nb1 != 0
        p1 = jnp.where(s1, nb1 & -nb1, ub1 & -ub1)
        nb2 = nb1 & ~p1
        ub2 = ub1 & ~p1
        s2 = nb2 != 0
        p2 = jnp.where(s2, nb2 & -nb2, ub2 & -ub2)

        d = dists_sc[...] - 1
        comb = comb_sc[...]
        eq0 = d == 0
        combf = jnp.sum(jnp.where(eq0, comb, 0), axis=0, keepdims=True)
        foundb = (combf & 1024) != 0
        tok_rand = tokr_sc[t, :][None, :]
        token = jnp.where(foundb, combf & 511, tok_rand)
        is_target = jnp.where(foundb, (combf >> 9) & 1, 0)
        eq0bits = jnp.where(foundb, 1 << (combf >> 11), 0)
        off = off_sc[t, :][None, :]
        occ = (d == off) | (d == off + 1) | (d == off + 2)
        occupied = jnp.max(occ.astype(jnp.int32), axis=0, keepdims=True)
        useb = (src_sc[t, :][None, :] == 1) & (~foundb) & (occupied == 0)
        pg0 = jnp.where(useb, p0, 0)
        pg1 = jnp.where(useb, p1, 0)
        pg2 = jnp.where(useb, p2, 0)
        m0 = (pg0 & bitw) != 0
        m1 = (pg1 & bitw) != 0
        m2 = (pg2 & bitw) != 0
        base_row = (1 << 10) | (rows << 11)
        dists_sc[...] = jnp.where(m0, off, jnp.where(m1, off + 1,
                                  jnp.where(m2, off + 2, d)))
        comb_sc[...] = jnp.where(m0, off | base_row, jnp.where(m1, V | base_row,
                                 jnp.where(m2, token | 512 | base_row, comb)))
        tokb_sc[t, :] = token[0, :]
        tmb_sc[t, :] = is_target[0, :]
        wb = pg0 | pg1 | pg2
        add0 = jnp.where(off == 0, pg0, 0)
        negnext = ((negbits | eq0bits) & ~wb) | add0
        return (nka, nkb, negnext)

    nb_in = negb_sc[0, :][None, :]
    kaf, kbf, nbf = lax.fori_loop(
        0, T, step_body, (key_sc[0, :], key_sc[1, :], nb_in), unroll=4)
    key_sc[0, :] = kaf
    key_sc[1, :] = kbf
    negb_sc[0, :] = nbf[0, :]

    tok_tr = tokb_sc[...].T
    tm_tr = tmb_sc[...].T
    tokens_ref[...] = tok_tr
    tmask_ref[...] = tm_tr == 1
    imask_ref[...] = jnp.ones((B, T), jnp.bool_)
    lanes = lax.broadcasted_iota(jnp.int32, (B, T, V + 1), 2)
    eq = lanes == tok_tr[:, :, None]
    base = jnp.where(lanes < V, jnp.float32(0.9), jnp.float32(0.1))
    probs_ref[...] = jnp.where(tm_tr[:, :, None] == 1,
                               eq.astype(jnp.float32), base)


@jax.jit
def kernel(key_B, fixed_offsets):
    del fixed_offsets
    kd = jax.random.key_data(key_B)
    kd_t = kd.T
    out_shapes = (
        jax.ShapeDtypeStruct((B, C), jnp.int32),
        jax.ShapeDtypeStruct((B, C, V + 1), jnp.float32),
        jax.ShapeDtypeStruct((B, C), jnp.bool_),
        jax.ShapeDtypeStruct((B, C), jnp.bool_),
    )
    tokens, probs, imask, tmask = pl.pallas_call(
        _kernel,
        grid=(NB,),
        in_specs=[pl.BlockSpec((2, B), lambda i: (0, 0))],
        out_specs=[
            pl.BlockSpec((B, T), lambda i: (0, i)),
            pl.BlockSpec((B, T, V + 1), lambda i: (0, i, 0)),
            pl.BlockSpec((B, T), lambda i: (0, i)),
            pl.BlockSpec((B, T), lambda i: (0, i)),
        ],
        out_shape=out_shapes,
        compiler_params=pltpu.CompilerParams(
            vmem_limit_bytes=100 * 1024 * 1024,
        ),
        scratch_shapes=[
            pltpu.VMEM((2, B), jnp.uint32),
            pltpu.VMEM((T, B), jnp.uint32),
            pltpu.VMEM((T, B), jnp.uint32),
            pltpu.VMEM((T, B), jnp.int32),
            pltpu.VMEM((T, B), jnp.int32),
            pltpu.VMEM((T, B), jnp.int32),
            pltpu.VMEM((T, B), jnp.int32),
            pltpu.VMEM((T, B), jnp.int32),
            pltpu.VMEM((SPAD, B), jnp.int32),
            pltpu.VMEM((SPAD, B), jnp.int32),
            pltpu.VMEM((1, B), jnp.int32),
        ],
    )(kd_t)
    return (kd, tokens, probs, imask, tmask)


# speedup vs baseline: 1.1286x; 1.1286x over previous
import jax
import jax.numpy as jnp
from jax import lax
from jax.experimental import pallas as pl
from jax.experimental.pallas import tpu as pltpu

B = 64
C = 2048
V = 256
NSLOT = 30
SPAD = 32
T = 128
NB = C // T

_U32 = jnp.uint32


def _tf(ka, kb, x0c, x1c):
    ks2 = ka ^ kb ^ _U32(0x1BD11BDA)
    x0 = ka + _U32(x0c)
    x1 = kb + _U32(x1c)
    rot = (13, 15, 26, 6, 17, 29, 16, 24)
    ks = (kb, ks2, ka)
    for i in range(5):
        for j in range(4):
            r = rot[(i % 2) * 4 + j]
            x0 = x0 + x1
            x1 = ((x1 << _U32(r)) | (x1 >> _U32(32 - r))) ^ x0
        x0 = x0 + ks[i % 3]
        x1 = x1 + ks[(i + 1) % 3] + _U32(i + 1)
    return x0, x1


def _mod10(x_u32):
    y = (x_u32 & _U32(0xFFFF)) + _U32(6) * (x_u32 >> _U32(16))
    yf = y.astype(jnp.float32)
    q = ((yf + 0.5) * 0.1).astype(jnp.int32)
    return y.astype(jnp.int32) - 10 * q


def _kernel(kd_ref,
            tokens_ref, probs_ref, imask_ref, tmask_ref,
            key_sc,
            keys_a, keys_b,
            tokr_sc, src_sc, off_sc,
            tokb_sc, tmb_sc,
            dists_sc, comb_sc,
            negb_sc):
    i = pl.program_id(0)

    rows = lax.broadcasted_iota(jnp.int32, (SPAD, B), 0)
    rows_real = rows < NSLOT

    @pl.when(i == 0)
    def _init():
        dists_sc[...] = jnp.where(rows_real, 0, -100000)
        comb_sc[...] = (1 << 10) | (rows << 11)
        negb_sc[...] = jnp.full((1, B), (1 << NSLOT) - 1, jnp.int32)

        def chain_body(t, carry):
            ka, kb = carry
            keys_a[t, :] = ka
            keys_b[t, :] = kb
            return _tf(ka, kb, 0, 3)

        kaf, kbf = lax.fori_loop(0, T, chain_body,
                                 (kd_ref[0, :], kd_ref[1, :]), unroll=4)
        key_sc[0, :] = kaf
        key_sc[1, :] = kbf

    ka = keys_a[...]
    kb = keys_b[...]
    k1a, k1b = _tf(ka, kb, 0, 0)
    k2a, k2b = _tf(ka, kb, 0, 1)
    k3a, k3b = _tf(ka, kb, 0, 2)
    s1a, s1b = _tf(k1a, k1b, 0, 1)
    la, lb = _tf(s1a, s1b, 0, 0)
    tokr_sc[...] = ((la ^ lb) & _U32(255)).astype(jnp.int32)
    s0a, s0b = _tf(k2a, k2b, 0, 0)
    s1a, s1b = _tf(k2a, k2b, 0, 1)
    ha, hb = _tf(s0a, s0b, 0, 0)
    la, lb = _tf(s1a, s1b, 0, 0)
    hi10 = _mod10(ha ^ hb)
    lo10 = _mod10(la ^ lb)
    v = hi10 * 6 + lo10
    vq = ((v.astype(jnp.float32) + 0.5) * 0.1).astype(jnp.int32)
    src_sc[...] = (v - 10 * vq == 0).astype(jnp.int32)
    s1a, s1b = _tf(k3a, k3b, 0, 1)
    la, lb = _tf(s1a, s1b, 0, 0)
    off_sc[...] = ((la ^ lb) & _U32(255)).astype(jnp.int32)
    bitw = jnp.where(rows_real, 1 << jnp.minimum(rows, 30), 0)
    REAL = (1 << NSLOT) - 1

    def step_body(t, carry):
        cka, ckb, negbits, dprev, comb = carry
        keys_a[t, :] = cka
        keys_b[t, :] = ckb
        nka, nkb = _tf(cka, ckb, 0, 3)

        nb0 = negbits
        ub0 = (~nb0) & REAL
        s0 = nb0 != 0
        p0 = jnp.where(s0, nb0 & -nb0, ub0 & -ub0)
        nb1 = nb0 & ~p0
        ub1 = ub0 & ~p0
        s1 = nb1 != 0
        p1 = jnp.where(s1, nb1 & -nb1, ub1 & -ub1)
        nb2 = nb1 & ~p1
        ub2 = ub1 & ~p1
        s2 = nb2 != 0
        p2 = jnp.where(s2, nb2 & -nb2, ub2 & -ub2)

        d = dprev - 1
        eq0 = d == 0
        combf = jnp.sum(jnp.where(eq0, comb, 0), axis=0, keepdims=True)
        foundb = (combf & 1024) != 0
        tok_rand = tokr_sc[t, :][None, :]
        token = jnp.where(foundb, combf & 511, tok_rand)
        is_target = jnp.where(foundb, (combf >> 9) & 1, 0)
        eq0bits = jnp.where(foundb, 1 << (combf >> 11), 0)
        off = off_sc[t, :][None, :]
        occ = (d == off) | (d == off + 1) | (d == off + 2)
        occupied = jnp.max(occ.astype(jnp.int32), axis=0, keepdims=True)
        useb = (src_sc[t, :][None, :] == 1) & (~foundb) & (occupied == 0)
        pg0 = jnp.where(useb, p0, 0)
        pg1 = jnp.where(useb, p1, 0)
        pg2 = jnp.where(useb, p2, 0)
        m0 = (pg0 & bitw) != 0
        m1 = (pg1 & bitw) != 0
        m2 = (pg2 & bitw) != 0
        base_row = (1 << 10) | (rows << 11)
        dnext = jnp.where(m0, off, jnp.where(m1, off + 1,
                          jnp.where(m2, off + 2, d)))
        combnext = jnp.where(m0, off | base_row, jnp.where(m1, V | base_row,
                             jnp.where(m2, token | 512 | base_row, comb)))
        tokb_sc[t, :] = token[0, :]
        tmb_sc[t, :] = is_target[0, :]
        wb = pg0 | pg1 | pg2
        add0 = jnp.where(off == 0, pg0, 0)
        negnext = ((negbits | eq0bits) & ~wb) | add0
        return (nka, nkb, negnext, dnext, combnext)

    nb_in = negb_sc[0, :][None, :]
    kaf, kbf, nbf, df, combf_out = lax.fori_loop(
        0, T, step_body,
        (key_sc[0, :], key_sc[1, :], nb_in, dists_sc[...], comb_sc[...]),
        unroll=4)
    key_sc[0, :] = kaf
    key_sc[1, :] = kbf
    negb_sc[0, :] = nbf[0, :]
    dists_sc[...] = df
    comb_sc[...] = combf_out

    tok_tr = tokb_sc[...].T
    tm_tr = tmb_sc[...].T
    tokens_ref[...] = tok_tr
    tmask_ref[...] = tm_tr == 1
    imask_ref[...] = jnp.ones((B, T), jnp.bool_)
    lanes = lax.broadcasted_iota(jnp.int32, (B, T, V + 1), 2)
    eq = lanes == tok_tr[:, :, None]
    base = jnp.where(lanes < V, jnp.float32(0.9), jnp.float32(0.1))
    probs_ref[...] = jnp.where(tm_tr[:, :, None] == 1,
                               eq.astype(jnp.float32), base)


@jax.jit
def kernel(key_B, fixed_offsets):
    del fixed_offsets
    kd = jax.random.key_data(key_B)
    kd_t = kd.T
    out_shapes = (
        jax.ShapeDtypeStruct((B, C), jnp.int32),
        jax.ShapeDtypeStruct((B, C, V + 1), jnp.float32),
        jax.ShapeDtypeStruct((B, C), jnp.bool_),
        jax.ShapeDtypeStruct((B, C), jnp.bool_),
    )
    tokens, probs, imask, tmask = pl.pallas_call(
        _kernel,
        grid=(NB,),
        in_specs=[pl.BlockSpec((2, B), lambda i: (0, 0))],
        out_specs=[
            pl.BlockSpec((B, T), lambda i: (0, i)),
            pl.BlockSpec((B, T, V + 1), lambda i: (0, i, 0)),
            pl.BlockSpec((B, T), lambda i: (0, i)),
            pl.BlockSpec((B, T), lambda i: (0, i)),
        ],
        out_shape=out_shapes,
        compiler_params=pltpu.CompilerParams(
            vmem_limit_bytes=100 * 1024 * 1024,
        ),
        scratch_shapes=[
            pltpu.VMEM((2, B), jnp.uint32),
            pltpu.VMEM((T, B), jnp.uint32),
            pltpu.VMEM((T, B), jnp.uint32),
            pltpu.VMEM((T, B), jnp.int32),
            pltpu.VMEM((T, B), jnp.int32),
            pltpu.VMEM((T, B), jnp.int32),
            pltpu.VMEM((T, B), jnp.int32),
            pltpu.VMEM((T, B), jnp.int32),
            pltpu.VMEM((SPAD, B), jnp.int32),
            pltpu.VMEM((SPAD, B), jnp.int32),
            pltpu.VMEM((1, B), jnp.int32),
        ],
    )(kd_t)
    return (kd, tokens, probs, imask, tmask)


# speedup vs baseline: 1.1292x; 1.0006x over previous
import jax
import jax.numpy as jnp
from jax import lax
from jax.experimental import pallas as pl
from jax.experimental.pallas import tpu as pltpu

B = 64
C = 2048
V = 256
NSLOT = 30
SPAD = 32
T = 128
NB = C // T

_U32 = jnp.uint32


def _tf(ka, kb, x0c, x1c):
    ks2 = ka ^ kb ^ _U32(0x1BD11BDA)
    x0 = ka + _U32(x0c)
    x1 = kb + _U32(x1c)
    rot = (13, 15, 26, 6, 17, 29, 16, 24)
    ks = (kb, ks2, ka)
    for i in range(5):
        for j in range(4):
            r = rot[(i % 2) * 4 + j]
            x0 = x0 + x1
            x1 = ((x1 << _U32(r)) | (x1 >> _U32(32 - r))) ^ x0
        x0 = x0 + ks[i % 3]
        x1 = x1 + ks[(i + 1) % 3] + _U32(i + 1)
    return x0, x1


def _mod10(x_u32):
    y = (x_u32 & _U32(0xFFFF)) + _U32(6) * (x_u32 >> _U32(16))
    yf = y.astype(jnp.float32)
    q = ((yf + 0.5) * 0.1).astype(jnp.int32)
    return y.astype(jnp.int32) - 10 * q


def _kernel(kd_ref,
            tokens_ref, probs_ref, imask_ref, tmask_ref,
            key_sc,
            keys_a, keys_b,
            tokr_sc,
            tokb_sc, tmb_sc,
            dists_sc, comb_sc,
            negb_sc):
    i = pl.program_id(0)

    rows = lax.broadcasted_iota(jnp.int32, (SPAD, B), 0)
    rows_real = rows < NSLOT

    @pl.when(i == 0)
    def _init():
        dists_sc[...] = jnp.where(rows_real, 0, -100000)
        comb_sc[...] = (1 << 10) | (rows << 11)
        negb_sc[...] = jnp.full((1, B), (1 << NSLOT) - 1, jnp.int32)

        def chain_body(t, carry):
            ka, kb = carry
            keys_a[t, :] = ka
            keys_b[t, :] = kb
            return _tf(ka, kb, 0, 3)

        kaf, kbf = lax.fori_loop(0, T, chain_body,
                                 (kd_ref[0, :], kd_ref[1, :]), unroll=4)
        key_sc[0, :] = kaf
        key_sc[1, :] = kbf

    ka = keys_a[...]
    kb = keys_b[...]
    k1a, k1b = _tf(ka, kb, 0, 0)
    k2a, k2b = _tf(ka, kb, 0, 1)
    k3a, k3b = _tf(ka, kb, 0, 2)
    s1a, s1b = _tf(k1a, k1b, 0, 1)
    la, lb = _tf(s1a, s1b, 0, 0)
    tok_r = ((la ^ lb) & _U32(255)).astype(jnp.int32)
    s0a, s0b = _tf(k2a, k2b, 0, 0)
    s1a, s1b = _tf(k2a, k2b, 0, 1)
    ha, hb = _tf(s0a, s0b, 0, 0)
    la, lb = _tf(s1a, s1b, 0, 0)
    hi10 = _mod10(ha ^ hb)
    lo10 = _mod10(la ^ lb)
    v = hi10 * 6 + lo10
    vq = ((v.astype(jnp.float32) + 0.5) * 0.1).astype(jnp.int32)
    src_r = (v - 10 * vq == 0).astype(jnp.int32)
    s1a, s1b = _tf(k3a, k3b, 0, 1)
    la, lb = _tf(s1a, s1b, 0, 0)
    off_r = ((la ^ lb) & _U32(255)).astype(jnp.int32)
    tokr_sc[...] = tok_r | (src_r << 8) | (off_r << 9)
    bitw = jnp.where(rows_real, 1 << jnp.minimum(rows, 30), 0)
    REAL = (1 << NSLOT) - 1

    def step_body(t, carry):
        cka, ckb, negbits, dprev, comb = carry
        keys_a[t, :] = cka
        keys_b[t, :] = ckb
        nka, nkb = _tf(cka, ckb, 0, 3)

        nb0 = negbits
        ub0 = (~nb0) & REAL
        s0 = nb0 != 0
        p0 = jnp.where(s0, nb0 & -nb0, ub0 & -ub0)
        nb1 = nb0 & ~p0
        ub1 = ub0 & ~p0
        s1 = nb1 != 0
        p1 = jnp.where(s1, nb1 & -nb1, ub1 & -ub1)
        nb2 = nb1 & ~p1
        ub2 = ub1 & ~p1
        s2 = nb2 != 0
        p2 = jnp.where(s2, nb2 & -nb2, ub2 & -ub2)

        d = dprev - 1
        eq0 = d == 0
        combf = jnp.sum(jnp.where(eq0, comb, 0), axis=0, keepdims=True)
        foundb = (combf & 1024) != 0
        draw = tokr_sc[t, :][None, :]
        token = jnp.where(foundb, combf & 511, draw & 255)
        is_target = jnp.where(foundb, (combf >> 9) & 1, 0)
        eq0bits = jnp.where(foundb, 1 << (combf >> 11), 0)
        off = draw >> 9
        occ = (d == off) | (d == off + 1) | (d == off + 2)
        occupied = jnp.max(occ.astype(jnp.int32), axis=0, keepdims=True)
        useb = ((draw & 256) != 0) & (~foundb) & (occupied == 0)
        pg0 = jnp.where(useb, p0, 0)
        pg1 = jnp.where(useb, p1, 0)
        pg2 = jnp.where(useb, p2, 0)
        m0 = (pg0 & bitw) != 0
        m1 = (pg1 & bitw) != 0
        m2 = (pg2 & bitw) != 0
        base_row = (1 << 10) | (rows << 11)
        dnext = jnp.where(m0, off, jnp.where(m1, off + 1,
                          jnp.where(m2, off + 2, d)))
        combnext = jnp.where(m0, off | base_row, jnp.where(m1, V | base_row,
                             jnp.where(m2, token | 512 | base_row, comb)))
        tokb_sc[t, :] = token[0, :]
        tmb_sc[t, :] = is_target[0, :]
        wb = pg0 | pg1 | pg2
        add0 = jnp.where(off == 0, pg0, 0)
        negnext = ((negbits | eq0bits) & ~wb) | add0
        return (nka, nkb, negnext, dnext, combnext)

    nb_in = negb_sc[0, :][None, :]
    kaf, kbf, nbf, df, combf_out = lax.fori_loop(
        0, T, step_body,
        (key_sc[0, :], key_sc[1, :], nb_in, dists_sc[...], comb_sc[...]),
        unroll=4)
    key_sc[0, :] = kaf
    key_sc[1, :] = kbf
    negb_sc[0, :] = nbf[0, :]
    dists_sc[...] = df
    comb_sc[...] = combf_out

    tok_tr = tokb_sc[...].T
    tm_tr = tmb_sc[...].T
    tokens_ref[...] = tok_tr
    tmask_ref[...] = tm_tr == 1
    imask_ref[...] = jnp.ones((B, T), jnp.bool_)
    lanes = lax.broadcasted_iota(jnp.int32, (B, T, V + 1), 2)
    eq = lanes == tok_tr[:, :, None]
    base = jnp.where(lanes < V, jnp.float32(0.9), jnp.float32(0.1))
    probs_ref[...] = jnp.where(tm_tr[:, :, None] == 1,
                               eq.astype(jnp.float32), base)


@jax.jit
def kernel(key_B, fixed_offsets):
    del fixed_offsets
    kd = jax.random.key_data(key_B)
    kd_t = kd.T
    out_shapes = (
        jax.ShapeDtypeStruct((B, C), jnp.int32),
        jax.ShapeDtypeStruct((B, C, V + 1), jnp.float32),
        jax.ShapeDtypeStruct((B, C), jnp.bool_),
        jax.ShapeDtypeStruct((B, C), jnp.bool_),
    )
    tokens, probs, imask, tmask = pl.pallas_call(
        _kernel,
        grid=(NB,),
        in_specs=[pl.BlockSpec((2, B), lambda i: (0, 0))],
        out_specs=[
            pl.BlockSpec((B, T), lambda i: (0, i)),
            pl.BlockSpec((B, T, V + 1), lambda i: (0, i, 0)),
            pl.BlockSpec((B, T), lambda i: (0, i)),
            pl.BlockSpec((B, T), lambda i: (0, i)),
        ],
        out_shape=out_shapes,
        compiler_params=pltpu.CompilerParams(
            vmem_limit_bytes=100 * 1024 * 1024,
        ),
        scratch_shapes=[
            pltpu.VMEM((2, B), jnp.uint32),
            pltpu.VMEM((T, B), jnp.uint32),
            pltpu.VMEM((T, B), jnp.uint32),
            pltpu.VMEM((T, B), jnp.int32),
            pltpu.VMEM((T, B), jnp.int32),
            pltpu.VMEM((T, B), jnp.int32),
            pltpu.VMEM((SPAD, B), jnp.int32),
            pltpu.VMEM((SPAD, B), jnp.int32),
            pltpu.VMEM((1, B), jnp.int32),
        ],
    )(kd_t)
    return (kd, tokens, probs, imask, tmask)


# speedup vs baseline: 1.1301x; 1.0008x over previous
import jax
import jax.numpy as jnp
from jax import lax
from jax.experimental import pallas as pl
from jax.experimental.pallas import tpu as pltpu

B = 64
C = 2048
V = 256
NSLOT = 30
SPAD = 32
T = 128
NB = C // T

_U32 = jnp.uint32


def _tf(ka, kb, x0c, x1c):
    ks2 = ka ^ kb ^ _U32(0x1BD11BDA)
    x0 = ka + _U32(x0c)
    x1 = kb + _U32(x1c)
    rot = (13, 15, 26, 6, 17, 29, 16, 24)
    ks = (kb, ks2, ka)
    for i in range(5):
        for j in range(4):
            r = rot[(i % 2) * 4 + j]
            x0 = x0 + x1
            x1 = ((x1 << _U32(r)) | (x1 >> _U32(32 - r))) ^ x0
        x0 = x0 + ks[i % 3]
        x1 = x1 + ks[(i + 1) % 3] + _U32(i + 1)
    return x0, x1


def _mod10(x_u32):
    y = (x_u32 & _U32(0xFFFF)) + _U32(6) * (x_u32 >> _U32(16))
    yf = y.astype(jnp.float32)
    q = ((yf + 0.5) * 0.1).astype(jnp.int32)
    return y.astype(jnp.int32) - 10 * q


def _kernel(kd_ref,
            tokens_ref, probs_ref, imask_ref, tmask_ref,
            key_sc,
            keys_a, keys_b,
            tokr_sc,
            tokb_sc, tmb_sc,
            dists_sc, comb_sc,
            negb_sc):
    i = pl.program_id(0)

    rows = lax.broadcasted_iota(jnp.int32, (SPAD, B), 0)
    rows_real = rows < NSLOT

    @pl.when(i == 0)
    def _init():
        dists_sc[...] = jnp.where(rows_real, 0, -100000)
        comb_sc[...] = (1 << 10) | (rows << 11)
        negb_sc[...] = jnp.full((1, B), (1 << NSLOT) - 1, jnp.int32)

        def chain_body(t, carry):
            ka, kb = carry
            keys_a[t, :] = ka
            keys_b[t, :] = kb
            return _tf(ka, kb, 0, 3)

        kaf, kbf = lax.fori_loop(0, T, chain_body,
                                 (kd_ref[0, :], kd_ref[1, :]), unroll=8)
        key_sc[0, :] = kaf
        key_sc[1, :] = kbf

    ka = keys_a[...]
    kb = keys_b[...]
    k1a, k1b = _tf(ka, kb, 0, 0)
    k2a, k2b = _tf(ka, kb, 0, 1)
    k3a, k3b = _tf(ka, kb, 0, 2)
    s1a, s1b = _tf(k1a, k1b, 0, 1)
    la, lb = _tf(s1a, s1b, 0, 0)
    tok_r = ((la ^ lb) & _U32(255)).astype(jnp.int32)
    s0a, s0b = _tf(k2a, k2b, 0, 0)
    s1a, s1b = _tf(k2a, k2b, 0, 1)
    ha, hb = _tf(s0a, s0b, 0, 0)
    la, lb = _tf(s1a, s1b, 0, 0)
    hi10 = _mod10(ha ^ hb)
    lo10 = _mod10(la ^ lb)
    v = hi10 * 6 + lo10
    vq = ((v.astype(jnp.float32) + 0.5) * 0.1).astype(jnp.int32)
    src_r = (v - 10 * vq == 0).astype(jnp.int32)
    s1a, s1b = _tf(k3a, k3b, 0, 1)
    la, lb = _tf(s1a, s1b, 0, 0)
    off_r = ((la ^ lb) & _U32(255)).astype(jnp.int32)
    tokr_sc[...] = tok_r | (src_r << 8) | (off_r << 9)
    bitw = jnp.where(rows_real, 1 << jnp.minimum(rows, 30), 0)
    REAL = (1 << NSLOT) - 1

    def step_body(t, carry):
        cka, ckb, negbits, dprev, comb = carry
        keys_a[t, :] = cka
        keys_b[t, :] = ckb
        nka, nkb = _tf(cka, ckb, 0, 3)

        nb0 = negbits
        ub0 = (~nb0) & REAL
        s0 = nb0 != 0
        p0 = jnp.where(s0, nb0 & -nb0, ub0 & -ub0)
        nb1 = nb0 & ~p0
        ub1 = ub0 & ~p0
        s1 = nb1 != 0
        p1 = jnp.where(s1, nb1 & -nb1, ub1 & -ub1)
        nb2 = nb1 & ~p1
        ub2 = ub1 & ~p1
        s2 = nb2 != 0
        p2 = jnp.where(s2, nb2 & -nb2, ub2 & -ub2)

        d = dprev - 1
        eq0 = d == 0
        combf = jnp.sum(jnp.where(eq0, comb, 0), axis=0, keepdims=True)
        foundb = (combf & 1024) != 0
        draw = tokr_sc[t, :][None, :]
        token = jnp.where(foundb, combf & 511, draw & 255)
        is_target = jnp.where(foundb, (combf >> 9) & 1, 0)
        eq0bits = jnp.where(foundb, 1 << (combf >> 11), 0)
        off = draw >> 9
        occ = (d == off) | (d == off + 1) | (d == off + 2)
        occupied = jnp.max(occ.astype(jnp.int32), axis=0, keepdims=True)
        useb = ((draw & 256) != 0) & (~foundb) & (occupied == 0)
        pg0 = jnp.where(useb, p0, 0)
        pg1 = jnp.where(useb, p1, 0)
        pg2 = jnp.where(useb, p2, 0)
        m0 = (pg0 & bitw) != 0
        m1 = (pg1 & bitw) != 0
        m2 = (pg2 & bitw) != 0
        base_row = (1 << 10) | (rows << 11)
        dnext = jnp.where(m0, off, jnp.where(m1, off + 1,
                          jnp.where(m2, off + 2, d)))
        combnext = jnp.where(m0, off | base_row, jnp.where(m1, V | base_row,
                             jnp.where(m2, token | 512 | base_row, comb)))
        tokb_sc[t, :] = token[0, :]
        tmb_sc[t, :] = is_target[0, :]
        wb = pg0 | pg1 | pg2
        add0 = jnp.where(off == 0, pg0, 0)
        negnext = ((negbits | eq0bits) & ~wb) | add0
        return (nka, nkb, negnext, dnext, combnext)

    nb_in = negb_sc[0, :][None, :]
    kaf, kbf, nbf, df, combf_out = lax.fori_loop(
        0, T, step_body,
        (key_sc[0, :], key_sc[1, :], nb_in, dists_sc[...], comb_sc[...]),
        unroll=8)
    key_sc[0, :] = kaf
    key_sc[1, :] = kbf
    negb_sc[0, :] = nbf[0, :]
    dists_sc[...] = df
    comb_sc[...] = combf_out

    tok_tr = tokb_sc[...].T
    tm_tr = tmb_sc[...].T
    tokens_ref[...] = tok_tr
    tmask_ref[...] = tm_tr == 1
    imask_ref[...] = jnp.ones((B, T), jnp.bool_)
    lanes = lax.broadcasted_iota(jnp.int32, (B, T, V + 1), 2)
    eq = lanes == tok_tr[:, :, None]
    base = jnp.where(lanes < V, jnp.float32(0.9), jnp.float32(0.1))
    probs_ref[...] = jnp.where(tm_tr[:, :, None] == 1,
                               eq.astype(jnp.float32), base)


@jax.jit
def kernel(key_B, fixed_offsets):
    del fixed_offsets
    kd = jax.random.key_data(key_B)
    kd_t = kd.T
    out_shapes = (
        jax.ShapeDtypeStruct((B, C), jnp.int32),
        jax.ShapeDtypeStruct((B, C, V + 1), jnp.float32),
        jax.ShapeDtypeStruct((B, C), jnp.bool_),
        jax.ShapeDtypeStruct((B, C), jnp.bool_),
    )
    tokens, probs, imask, tmask = pl.pallas_call(
        _kernel,
        grid=(NB,),
        in_specs=[pl.BlockSpec((2, B), lambda i: (0, 0))],
        out_specs=[
            pl.BlockSpec((B, T), lambda i: (0, i)),
            pl.BlockSpec((B, T, V + 1), lambda i: (0, i, 0)),
            pl.BlockSpec((B, T), lambda i: (0, i)),
            pl.BlockSpec((B, T), lambda i: (0, i)),
        ],
        out_shape=out_shapes,
        compiler_params=pltpu.CompilerParams(
            vmem_limit_bytes=100 * 1024 * 1024,
        ),
        scratch_shapes=[
            pltpu.VMEM((2, B), jnp.uint32),
            pltpu.VMEM((T, B), jnp.uint32),
            pltpu.VMEM((T, B), jnp.uint32),
            pltpu.VMEM((T, B), jnp.int32),
            pltpu.VMEM((T, B), jnp.int32),
            pltpu.VMEM((T, B), jnp.int32),
            pltpu.VMEM((SPAD, B), jnp.int32),
            pltpu.VMEM((SPAD, B), jnp.int32),
            pltpu.VMEM((1, B), jnp.int32),
        ],
    )(kd_t)
    return (kd, tokens, probs, imask, tmask)
